# baseline (speedup 1.0000x reference)
import jax
import jax.numpy as jnp
from jax import lax
from jax.experimental import pallas as pl
from jax.experimental.pallas import tpu as pltpu
from jax.experimental.pallas import tpu_sc as plsc

N = 4096
T = 2048
LANES = 128
NTILES = T // LANES
NC = 2
NS = 16
NW = NC * NS
ROWS_PER_W = N // NW
REM_TILES = NTILES - 1
HALF_ROWS = ROWS_PER_W // 2
TAN_PI_8 = 0.4142135623730951
FULL = 0xFFFF


def _bin_bits(x, y):
    ax = jnp.abs(x)
    ay = jnp.abs(y)
    c1 = ay > ax * TAN_PI_8
    c2 = ay >= ax
    c3 = ay * TAN_PI_8 > ax
    p = jnp.where(c3, 8, jnp.where(c2, 4, jnp.where(c1, 2, 1)))
    p = jnp.where(x < 0.0, p << 4, p)
    return jnp.where(y < 0.0, p << 8, p)


def _sc_body(x_hbm, out_hbm, buf, rembuf, acc, sem0, sem1):
    c = lax.axis_index("c")
    s = lax.axis_index("s")
    wid = s * NC + c
    base = wid * ROWS_PER_W

    cp0 = pltpu.async_copy(
        x_hbm.at[pl.ds(base, HALF_ROWS), 0, :, :],
        buf.at[pl.ds(0, HALF_ROWS)], sem0)
    cp1 = pltpu.async_copy(
        x_hbm.at[pl.ds(base + HALF_ROWS, HALF_ROWS), 0, :, :],
        buf.at[pl.ds(HALF_ROWS, HALF_ROWS)], sem1)

    acc[...] = jnp.zeros((16,), jnp.int32)
    iota = lax.iota(jnp.int32, 16)
    rot_perms = [jnp.bitwise_and(iota + k, 15) for k in (8, 4, 2, 1)]

    def _or_all_lanes(m):
        for p in rot_perms:
            m = m | m.at[p].get(mode="promise_in_bounds")
        return m

    def _prefix_mask(r):
        m = jnp.full((16,), 0, jnp.int32)
        for k in range(LANES // 16):
            xv = buf[r, 0, pl.ds(k * 16, 16)]
            yv = buf[r, 1, pl.ds(k * 16, 16)]
            m = m | _bin_bits(xv, yv)
        return _or_all_lanes(m)

    def row_body(r, carry):
        and_m, accv = carry
        m_all = _prefix_mask(r)
        return and_m & m_all, accv + ((m_all >> iota) & 1)

    def half(lo):
        and_m, accv = lax.fori_loop(
            lo, lo + HALF_ROWS, row_body,
            (jnp.full((16,), -1, jnp.int32), jnp.zeros((16,), jnp.int32)))
        acc[...] = acc[...] + accv

        @pl.when(and_m[0] != FULL)
        def _rescan():
            def rescan_row(r, carry2):
                m_all = _prefix_mask(r)

                @pl.when(m_all[0] != FULL)
                def _finish_row():
                    pltpu.sync_copy(
                        x_hbm.at[base + r, pl.ds(1, REM_TILES), :, :],
                        rembuf)

                    def rem_body(jj, mm):
                        for k in range(LANES // 16):
                            xv = rembuf[jj, 0, pl.ds(k * 16, 16)]
                            yv = rembuf[jj, 1, pl.ds(k * 16, 16)]
                            mm = mm | _bin_bits(xv, yv)
                        return mm

                    m2 = _or_all_lanes(
                        lax.fori_loop(0, REM_TILES, rem_body, m_all))
                    acc[...] = acc[...] + (((m2 >> iota) & 1)
                                           - ((m_all >> iota) & 1))

                return carry2

            lax.fori_loop(lo, lo + HALF_ROWS, rescan_row, 0)

    cp0.wait()
    half(0)
    cp1.wait()
    half(HALF_ROWS)

    pltpu.sync_copy(acc, out_hbm.at[wid])


def kernel(c_seq):
    x4 = c_seq.reshape(N, NTILES, LANES, 2).transpose(0, 1, 3, 2)
    mesh = plsc.VectorSubcoreMesh(core_axis_name="c", subcore_axis_name="s")
    partial_occ = pl.kernel(
        _sc_body,
        out_type=jax.ShapeDtypeStruct((NW, 16), jnp.int32),
        mesh=mesh,
        scratch_types=[
            pltpu.VMEM((ROWS_PER_W, 2, LANES), jnp.float32),
            pltpu.VMEM((REM_TILES, 2, LANES), jnp.float32),
            pltpu.VMEM((16,), jnp.int32),
            pltpu.SemaphoreType.DMA,
            pltpu.SemaphoreType.DMA,
        ],
        compiler_params=pltpu.CompilerParams(use_tc_tiling_on_sc=False),
    )(x4)
    total = jnp.sum(partial_occ).astype(jnp.float32)
    return jnp.float32(1.0) - total / jnp.float32(N * 16)


# speedup vs baseline: 1.2585x; 1.2585x over previous
import jax
import jax.numpy as jnp
from jax import lax
from jax.experimental import pallas as pl
from jax.experimental.pallas import tpu as pltpu
from jax.experimental.pallas import tpu_sc as plsc

N = 4096
T = 2048
LANES = 128
NTILES = T // LANES
NC = 2
NS = 16
NW = NC * NS
ROWS_PER_W = N // NW
REM_TILES = NTILES - 1
HALF_ROWS = ROWS_PER_W // 2
TAN_PI_8 = 0.4142135623730951
FULL = 0xFFFF


def _bin_bits(x, y):
    ax = jnp.abs(x)
    ay = jnp.abs(y)
    c1 = ay > ax * TAN_PI_8
    c2 = ay >= ax
    c3 = ay * TAN_PI_8 > ax
    p = jnp.where(c3, 8, jnp.where(c2, 4, jnp.where(c1, 2, 1)))
    p = jnp.where(x < 0.0, p << 4, p)
    return jnp.where(y < 0.0, p << 8, p)


def _sc_body(x_hbm, out_hbm, buf, rembuf, acc, sem0, sem1):
    c = lax.axis_index("c")
    s = lax.axis_index("s")
    wid = s * NC + c
    base = wid * ROWS_PER_W

    cp0 = pltpu.async_copy(
        x_hbm.at[pl.ds(base, HALF_ROWS), 0, :, :],
        buf.at[pl.ds(0, HALF_ROWS)], sem0)
    cp1 = pltpu.async_copy(
        x_hbm.at[pl.ds(base + HALF_ROWS, HALF_ROWS), 0, :, :],
        buf.at[pl.ds(HALF_ROWS, HALF_ROWS)], sem1)

    acc[...] = jnp.zeros((16,), jnp.int32)
    iota = lax.iota(jnp.int32, 16)
    rot_perms = [jnp.bitwise_and(iota + k, 15) for k in (8, 4, 2, 1)]

    def _or_all_lanes(m):
        for p in rot_perms:
            m = m | m.at[p].get(mode="promise_in_bounds")
        return m

    def row_body(r, accv):
        m = jnp.full((16,), 0, jnp.int32)
        for k in range(LANES // 16):
            xv = buf[r, 0, pl.ds(k * 16, 16)]
            yv = buf[r, 1, pl.ds(k * 16, 16)]
            m = m | _bin_bits(xv, yv)
        m_all = _or_all_lanes(m)
        accv = accv + ((m_all >> iota) & 1)

        @pl.when(m_all[0] != FULL)
        def _fallback():
            pltpu.sync_copy(
                x_hbm.at[base + r, pl.ds(1, REM_TILES), :, :], rembuf)

            def rem_body(jj, mm):
                for k in range(LANES // 16):
                    xv = rembuf[jj, 0, pl.ds(k * 16, 16)]
                    yv = rembuf[jj, 1, pl.ds(k * 16, 16)]
                    mm = mm | _bin_bits(xv, yv)
                return mm

            m2 = _or_all_lanes(lax.fori_loop(0, REM_TILES, rem_body, m_all))
            acc[...] = acc[...] + (((m2 >> iota) & 1) - ((m_all >> iota) & 1))

        return accv

    cp0.wait()
    accv = lax.fori_loop(0, HALF_ROWS, row_body,
                         jnp.zeros((16,), jnp.int32))
    cp1.wait()
    accv = lax.fori_loop(HALF_ROWS, ROWS_PER_W, row_body, accv)

    acc[...] = acc[...] + accv
    pltpu.sync_copy(acc, out_hbm.at[wid])


def kernel(c_seq):
    x4 = c_seq.reshape(N, NTILES, LANES, 2).transpose(0, 1, 3, 2)
    mesh = plsc.VectorSubcoreMesh(core_axis_name="c", subcore_axis_name="s")
    partial_occ = pl.kernel(
        _sc_body,
        out_type=jax.ShapeDtypeStruct((NW, 16), jnp.int32),
        mesh=mesh,
        scratch_types=[
            pltpu.VMEM((ROWS_PER_W, 2, LANES), jnp.float32),
            pltpu.VMEM((REM_TILES, 2, LANES), jnp.float32),
            pltpu.VMEM((16,), jnp.int32),
            pltpu.SemaphoreType.DMA,
            pltpu.SemaphoreType.DMA,
        ],
        compiler_params=pltpu.CompilerParams(use_tc_tiling_on_sc=False),
    )(x4)
    total = jnp.sum(partial_occ).astype(jnp.float32)
    return jnp.float32(1.0) - total / jnp.float32(N * 16)


# speedup vs baseline: 1.2907x; 1.0255x over previous
import jax
import jax.numpy as jnp
from jax import lax
from jax.experimental import pallas as pl
from jax.experimental.pallas import tpu as pltpu
from jax.experimental.pallas import tpu_sc as plsc

N = 4096
T = 2048
LANES = 128
NTILES = T // LANES
NC = 2
NS = 16
NW = NC * NS
ROWS_PER_W = N // NW
REM_TILES = NTILES - 1
HALF_ROWS = ROWS_PER_W // 2
TAN_PI_8 = 0.4142135623730951
FULL = 0xFFFF


def _bin_bits(x, y):
    ax = jnp.abs(x)
    ay = jnp.abs(y)
    c1 = ay > ax * TAN_PI_8
    c2 = ay >= ax
    c3 = ay * TAN_PI_8 > ax
    p = jnp.where(c3, 8, jnp.where(c2, 4, jnp.where(c1, 2, 1)))
    p = jnp.where(x < 0.0, p << 4, p)
    return jnp.where(y < 0.0, p << 8, p)


def _sc_body(x_hbm, out_hbm, buf, rembuf, acc, sem0, sem1):
    c = lax.axis_index("c")
    s = lax.axis_index("s")
    wid = s * NC + c
    base = wid * ROWS_PER_W

    cp0 = pltpu.async_copy(
        x_hbm.at[pl.ds(base, HALF_ROWS), 0, :, :],
        buf.at[pl.ds(0, HALF_ROWS)], sem0)
    cp1 = pltpu.async_copy(
        x_hbm.at[pl.ds(base + HALF_ROWS, HALF_ROWS), 0, :, :],
        buf.at[pl.ds(HALF_ROWS, HALF_ROWS)], sem1)

    acc[...] = jnp.zeros((16,), jnp.int32)
    iota = lax.iota(jnp.int32, 16)
    rot_perms = [jnp.bitwise_and(iota + k, 15) for k in (8, 4, 2, 1)]

    def _or_all_lanes(m):
        for p in rot_perms:
            m = m | m.at[p].get(mode="promise_in_bounds")
        return m

    def _prefix_mask(r):
        m = jnp.full((16,), 0, jnp.int32)
        for k in range(LANES // 16):
            xv = buf[r, 0, pl.ds(k * 16, 16)]
            yv = buf[r, 1, pl.ds(k * 16, 16)]
            m = m | _bin_bits(xv, yv)
        return _or_all_lanes(m)

    def _finish_row(r, m_all):
        pltpu.sync_copy(
            x_hbm.at[base + r, pl.ds(1, REM_TILES), :, :], rembuf)

        def rem_body(jj, mm):
            for k in range(LANES // 16):
                xv = rembuf[jj, 0, pl.ds(k * 16, 16)]
                yv = rembuf[jj, 1, pl.ds(k * 16, 16)]
                mm = mm | _bin_bits(xv, yv)
            return mm

        m2 = _or_all_lanes(lax.fori_loop(0, REM_TILES, rem_body, m_all))
        acc[...] = acc[...] + (((m2 >> iota) & 1) - ((m_all >> iota) & 1))

    def pair_body(i, accv):
        r = 2 * i
        ma = _prefix_mask(r)
        mb = _prefix_mask(r + 1)
        accv = accv + ((ma >> iota) & 1) + ((mb >> iota) & 1)
        both = ma & mb

        @pl.when(both[0] != FULL)
        def _rare():
            def fix_fn(rr, carry2):
                mx = _prefix_mask(rr)

                @pl.when(mx[0] != FULL)
                def _fb():
                    _finish_row(rr, mx)

                return carry2

            lax.fori_loop(r, r + 2, fix_fn, 0)

        return accv

    cp0.wait()
    accv = lax.fori_loop(0, HALF_ROWS // 2, pair_body,
                         jnp.zeros((16,), jnp.int32))
    cp1.wait()
    accv = lax.fori_loop(HALF_ROWS // 2, ROWS_PER_W // 2, pair_body, accv)

    acc[...] = acc[...] + accv
    pltpu.sync_copy(acc, out_hbm.at[wid])


def kernel(c_seq):
    x4 = c_seq.reshape(N, NTILES, LANES, 2).transpose(0, 1, 3, 2)
    mesh = plsc.VectorSubcoreMesh(core_axis_name="c", subcore_axis_name="s")
    partial_occ = pl.kernel(
        _sc_body,
        out_type=jax.ShapeDtypeStruct((NW, 16), jnp.int32),
        mesh=mesh,
        scratch_types=[
            pltpu.VMEM((ROWS_PER_W, 2, LANES), jnp.float32),
            pltpu.VMEM((REM_TILES, 2, LANES), jnp.float32),
            pltpu.VMEM((16,), jnp.int32),
            pltpu.SemaphoreType.DMA,
            pltpu.SemaphoreType.DMA,
        ],
        compiler_params=pltpu.CompilerParams(use_tc_tiling_on_sc=False),
    )(x4)
    total = jnp.sum(partial_occ).astype(jnp.float32)
    return jnp.float32(1.0) - total / jnp.float32(N * 16)
